# baseline (speedup 1.0000x reference)
import functools

import jax
import jax.numpy as jnp
from jax import lax
from jax.experimental import pallas as pl
from jax.experimental.pallas import tpu as pltpu
from jax.experimental.pallas import tpu_sc as plsc

B, C, H, W = 64, 8, 128, 128
HW = H * W
M = 128
N = 128
NTILES = 32
BPT = B // NTILES
EPT = BPT * M * C
NVREG = EPT // 16
NCHUNK = EPT // 128

_mesh = plsc.VectorSubcoreMesh(core_axis_name="c", subcore_axis_name="s")


@functools.partial(
    pl.kernel,
    out_type=jax.ShapeDtypeStruct((NTILES, 16), jnp.float32),
    mesh=_mesh,
    compiler_params=pltpu.CompilerParams(needs_layout_passes=False),
    scratch_types=[
        pltpu.VMEM((BPT * M,), jnp.int32),
        pltpu.VMEM((BPT * N,), jnp.int32),
        pltpu.VMEM((BPT * 4 * M,), jnp.int32),
        pltpu.VMEM((BPT * M,), jnp.int32),
        pltpu.VMEM((BPT * N,), jnp.int32),
        pltpu.VMEM((EPT,), jnp.float32),
        pltpu.VMEM((EPT,), jnp.float32),
        pltpu.VMEM((EPT,), jnp.float32),
        pltpu.VMEM((EPT,), jnp.float32),
        pltpu.VMEM((16,), jnp.float32),
        pltpu.SemaphoreType.DMA,
        pltpu.SemaphoreType.DMA,
        pltpu.SemaphoreType.DMA,
        pltpu.SemaphoreType.DMA,
    ],
)
def _pair_loss_sc(o1_hbm, o2_hbm, i1_hbm, i2_hbm, mk_hbm, mkc_hbm, ctr_hbm,
                  t1_hbm, t2_hbm, out_hbm,
                  ind1_v, ind2_v, ctr_v, mk_v, mkc_v, t1_v, t2_v,
                  pred1_v, pred2_v, part_v, sem, semb, semc, semd):
    wid = lax.axis_index("s") * 2 + lax.axis_index("c")
    b0 = wid * BPT

    c1 = pltpu.async_copy(i1_hbm.at[pl.ds(b0 * M, BPT * M)], ind1_v, sem)
    c2 = pltpu.async_copy(i2_hbm.at[pl.ds(b0 * N, BPT * N)], ind2_v, sem)
    c1.wait()
    c2.wait()

    iota = lax.iota(jnp.int32, 16)

    sems = (sem, semb, semc, semd)
    groups = ([], [], [], [])
    for j in range(NCHUNK):
        bb = j >> 3
        ch = j & 7
        base = ((b0 + bb) * C + ch) * HW
        i1row = ind1_v.at[pl.ds(bb * M, M)]
        i2row = ind2_v.at[pl.ds(bb * N, N)]
        grp, gsem = groups[j >> 2], sems[j >> 2]
        grp.append(pltpu.async_copy(
            o1_hbm.at[pl.ds(base, HW)].at[i1row],
            pred1_v.at[pl.ds(j * 128, 128)], gsem))
        grp.append(pltpu.async_copy(
            o2_hbm.at[pl.ds(base, HW)].at[i2row],
            pred2_v.at[pl.ds(j * 128, 128)], gsem))
    copies = groups[0]

    ctr_base = (b0 >> 3) * (4 * 8 * 128) + (b0 & 7) * 128
    for tc in range(4):
        copies.append(pltpu.async_copy(
            ctr_hbm.at[pl.ds(ctr_base + tc * (8 * 128), BPT * 128)],
            ctr_v.at[pl.ds(tc * BPT * 128, BPT * 128)], sem))
    copies.append(pltpu.async_copy(
        mk_hbm.at[pl.ds(b0 * M, BPT * M)], mk_v, sem))
    copies.append(pltpu.async_copy(
        mkc_hbm.at[pl.ds(b0 * N, BPT * N)], mkc_v, sem))
    copies.append(pltpu.async_copy(
        t1_hbm.at[pl.ds(b0 * M * C, EPT)], t1_v, sem))
    copies.append(pltpu.async_copy(
        t2_hbm.at[pl.ds(b0 * N * C, EPT)], t2_v, sem))

    for cp in copies:
        cp.wait()

    zero = jnp.zeros((16,), jnp.float32)

    def _body(i, carry):
        a1, a2, a3, ad = carry
        bb = i >> 6
        ch = (i >> 3) & 7
        m0 = (i & 7) * 16
        mrow = pl.ds(bb * M + m0, 16)
        mkf = mk_v[mrow].astype(jnp.float32)
        mc = mkc_v[mrow]
        jv = ((4 * m0) >> 7) * (BPT * 128) + bb * 128 + ((4 * m0) & 127) \
            + (ch >> 1) + 4 * iota
        cv = plsc.load_gather(ctr_v, [jv])
        q = 2 * cv + (ch & 1)
        src = bb * (N * C) + (q & 7) * M + (q >> 3)
        p2g = plsc.load_gather(pred2_v, [src])
        t2g = plsc.load_gather(t2_v, [src])
        sl = pl.ds(i * 16, 16)
        p1 = pred1_v[sl]
        t1 = t1_v[sl]
        p2 = pred2_v[sl]
        t2 = t2_v[sl]
        delta = (jnp.abs(p1 - t1) + jnp.abs(p2g - t2g)) / (jnp.abs(t1) + 0.0001)
        delta = delta * delta
        dm = jnp.where(delta > 1.0, 0.0, 1.0)
        delta = delta * dm + (1.0 - dm)
        wgt = 1.0 - jnp.exp(-3.14 * delta)
        mw = mkf * wgt
        a1 = a1 + jnp.abs(p1 * mw - t1 * mw)
        a2 = a2 + jnp.abs(p2g * mw - t2g * mw)
        big = jnp.where((t2 == 0.0).astype(jnp.int32) == mc, 1.0, 0.0)
        a3 = a3 + jnp.abs(p2 * big - t2 * big)
        ad = ad + mkf
        return a1, a2, a3, ad

    carry = (zero, zero, zero, zero)
    Q = NVREG // 4
    for g in range(4):
        if g > 0:
            for cp in groups[g]:
                cp.wait()
        carry = pl.loop(g * Q, (g + 1) * Q, init_carry=carry,
                        unroll=2)(_body)
    a1, a2, a3, ad = carry
    s1 = jnp.sum(a1)
    s2 = jnp.sum(a2)
    s3 = jnp.sum(a3)
    sd = jnp.sum(ad)
    part = (jnp.where(iota == 0, s1, 0.0) + jnp.where(iota == 1, s2, 0.0)
            + jnp.where(iota == 2, s3, 0.0) + jnp.where(iota == 3, sd, 0.0))
    part_v[...] = part
    pltpu.sync_copy(part_v, out_hbm.at[wid])


def kernel(output1, ind1, output2, ind2, mask, mask_cro, ctr_cro_ind,
           target1, target2, hm_ctxy):
    del hm_ctxy
    parts = _pair_loss_sc(
        output1.reshape(B * C * HW),
        output2.reshape(B * C * HW),
        ind1.reshape(B * M),
        ind2.reshape(B * N),
        mask.reshape(B * M),
        mask_cro.reshape(B * N),
        ctr_cro_ind.reshape(B // 8, 8, 4, 128).transpose(0, 2, 1, 3)
        .reshape(B * 4 * M),
        jnp.transpose(target1, (0, 2, 1)).reshape(B * M * C),
        jnp.transpose(target2, (0, 2, 1)).reshape(B * N * C),
    )
    s = jnp.sum(parts, axis=0)
    denom = s[3] + 0.0001
    loss1 = s[0] / denom
    loss2 = s[1] / denom
    loss3 = s[2] / denom
    return (loss1, 0.5 * loss2 + 0.2 * loss3)


# speedup vs baseline: 1.0201x; 1.0201x over previous
import functools

import jax
import jax.numpy as jnp
from jax import lax
from jax.experimental import pallas as pl
from jax.experimental.pallas import tpu as pltpu
from jax.experimental.pallas import tpu_sc as plsc

B, C, H, W = 64, 8, 128, 128
HW = H * W
M = 128
N = 128
NTILES = 32
BPT = B // NTILES
EPT = BPT * M * C
NVREG = EPT // 16
NCHUNK = EPT // 128

_mesh = plsc.VectorSubcoreMesh(core_axis_name="c", subcore_axis_name="s")


@functools.partial(
    pl.kernel,
    out_type=jax.ShapeDtypeStruct((NTILES, 16), jnp.float32),
    mesh=_mesh,
    compiler_params=pltpu.CompilerParams(needs_layout_passes=False),
    scratch_types=[
        pltpu.VMEM((BPT * M,), jnp.int32),
        pltpu.VMEM((BPT * N,), jnp.int32),
        pltpu.VMEM((BPT * 4 * M,), jnp.int32),
        pltpu.VMEM((BPT * M,), jnp.int32),
        pltpu.VMEM((BPT * N,), jnp.int32),
        pltpu.VMEM((EPT,), jnp.float32),
        pltpu.VMEM((EPT,), jnp.float32),
        pltpu.VMEM((EPT,), jnp.float32),
        pltpu.VMEM((EPT,), jnp.float32),
        pltpu.VMEM((16,), jnp.float32),
        pltpu.SemaphoreType.DMA,
        pltpu.SemaphoreType.DMA,
    ],
)
def _pair_loss_sc(o1_hbm, o2_hbm, i1_hbm, i2_hbm, mk_hbm, mkc_hbm, ctr_hbm,
                  t1_hbm, t2_hbm, out_hbm,
                  ind1_v, ind2_v, ctr_v, mk_v, mkc_v, t1_v, t2_v,
                  pred1_v, pred2_v, part_v, sem, semb):
    wid = lax.axis_index("s") * 2 + lax.axis_index("c")
    b0 = wid * BPT

    c1 = pltpu.async_copy(i1_hbm.at[pl.ds(b0 * M, BPT * M)], ind1_v, sem)
    c2 = pltpu.async_copy(i2_hbm.at[pl.ds(b0 * N, BPT * N)], ind2_v, sem)
    c1.wait()
    c2.wait()

    iota = lax.iota(jnp.int32, 16)

    copies = []
    copies_b1 = []
    for j in range(NCHUNK):
        bb = j >> 3
        ch = j & 7
        base = ((b0 + bb) * C + ch) * HW
        i1row = ind1_v.at[pl.ds(bb * M, M)]
        i2row = ind2_v.at[pl.ds(bb * N, N)]
        grp, gsem = (copies, sem) if bb == 0 else (copies_b1, semb)
        grp.append(pltpu.async_copy(
            o1_hbm.at[pl.ds(base, HW)].at[i1row],
            pred1_v.at[pl.ds(j * 128, 128)], gsem))
        grp.append(pltpu.async_copy(
            o2_hbm.at[pl.ds(base, HW)].at[i2row],
            pred2_v.at[pl.ds(j * 128, 128)], gsem))

    ctr_base = (b0 >> 3) * (4 * 8 * 128) + (b0 & 7) * 128
    for tc in range(4):
        copies.append(pltpu.async_copy(
            ctr_hbm.at[pl.ds(ctr_base + tc * (8 * 128), BPT * 128)],
            ctr_v.at[pl.ds(tc * BPT * 128, BPT * 128)], sem))
    copies.append(pltpu.async_copy(
        mk_hbm.at[pl.ds(b0 * M, BPT * M)], mk_v, sem))
    copies.append(pltpu.async_copy(
        mkc_hbm.at[pl.ds(b0 * N, BPT * N)], mkc_v, sem))
    copies.append(pltpu.async_copy(
        t1_hbm.at[pl.ds(b0 * M * C, EPT)], t1_v, sem))
    copies.append(pltpu.async_copy(
        t2_hbm.at[pl.ds(b0 * N * C, EPT)], t2_v, sem))

    for cp in copies:
        cp.wait()

    zero = jnp.zeros((16,), jnp.float32)

    def _body(i, carry):
        a1, a2, a3, ad = carry
        bb = i >> 6
        ch = (i >> 3) & 7
        m0 = (i & 7) * 16
        mrow = pl.ds(bb * M + m0, 16)
        mkf = mk_v[mrow].astype(jnp.float32)
        mc = mkc_v[mrow]
        jv = ((4 * m0) >> 7) * (BPT * 128) + bb * 128 + ((4 * m0) & 127) \
            + (ch >> 1) + 4 * iota
        cv = plsc.load_gather(ctr_v, [jv])
        q = 2 * cv + (ch & 1)
        src = bb * (N * C) + (q & 7) * M + (q >> 3)
        p2g = plsc.load_gather(pred2_v, [src])
        t2g = plsc.load_gather(t2_v, [src])
        sl = pl.ds(i * 16, 16)
        p1 = pred1_v[sl]
        t1 = t1_v[sl]
        p2 = pred2_v[sl]
        t2 = t2_v[sl]
        delta = (jnp.abs(p1 - t1) + jnp.abs(p2g - t2g)) / (jnp.abs(t1) + 0.0001)
        delta = delta * delta
        dm = jnp.where(delta > 1.0, 0.0, 1.0)
        delta = delta * dm + (1.0 - dm)
        wgt = 1.0 - jnp.exp(-3.14 * delta)
        mw = mkf * wgt
        a1 = a1 + jnp.abs(p1 * mw - t1 * mw)
        a2 = a2 + jnp.abs(p2g * mw - t2g * mw)
        big = jnp.where((t2 == 0.0).astype(jnp.int32) == mc, 1.0, 0.0)
        a3 = a3 + jnp.abs(p2 * big - t2 * big)
        ad = ad + mkf
        return a1, a2, a3, ad

    carry0 = pl.loop(0, NVREG // 2, init_carry=(zero, zero, zero, zero),
                     unroll=2)(_body)

    for cp in copies_b1:
        cp.wait()

    a1, a2, a3, ad = pl.loop(NVREG // 2, NVREG, init_carry=carry0,
                             unroll=2)(_body)
    s1 = jnp.sum(a1)
    s2 = jnp.sum(a2)
    s3 = jnp.sum(a3)
    sd = jnp.sum(ad)
    part = (jnp.where(iota == 0, s1, 0.0) + jnp.where(iota == 1, s2, 0.0)
            + jnp.where(iota == 2, s3, 0.0) + jnp.where(iota == 3, sd, 0.0))
    part_v[...] = part
    pltpu.sync_copy(part_v, out_hbm.at[wid])


def kernel(output1, ind1, output2, ind2, mask, mask_cro, ctr_cro_ind,
           target1, target2, hm_ctxy):
    del hm_ctxy
    parts = _pair_loss_sc(
        output1.reshape(B * C * HW),
        output2.reshape(B * C * HW),
        ind1.reshape(B * M),
        ind2.reshape(B * N),
        mask.reshape(B * M),
        mask_cro.reshape(B * N),
        ctr_cro_ind.reshape(B // 8, 8, 4, 128).transpose(0, 2, 1, 3)
        .reshape(B * 4 * M),
        jnp.transpose(target1, (0, 2, 1)).reshape(B * M * C),
        jnp.transpose(target2, (0, 2, 1)).reshape(B * N * C),
    )
    s = jnp.sum(parts, axis=0)
    denom = s[3] + 0.0001
    loss1 = s[0] / denom
    loss2 = s[1] / denom
    loss3 = s[2] / denom
    return (loss1, 0.5 * loss2 + 0.2 * loss3)
